# Initial kernel scaffold; baseline (speedup 1.0000x reference)
import jax
import jax.numpy as jnp
from jax.experimental import pallas as pl


def kernel(node_features, edge_features, Esrc, Etgt, batch, W_in, b_in, W_ee, b_ee, Wg0, bg0, Wg1, bg1, Wg2, bg2, Wi, Wh, bi, bh, W_out, b_out):
    raise NotImplementedError("write your pallas kernel here")


# speedup vs baseline: 4.5089x; 4.5089x over previous
import functools

import jax
import jax.numpy as jnp
from jax import lax
from jax.experimental import pallas as pl
from jax.experimental.pallas import tpu as pltpu
from jax.experimental.pallas import tpu_sc as plsc

_NUM_GRAPHS = 64
_STEPS = 12


@functools.cache
def _make_gc_sc(n, e, h):
    info = plsc.get_sparse_core_info()
    nc, ns = info.num_cores, info.num_subcores
    nw = nc * ns
    ch = 128
    assert e % ch == 0
    total_chunks = e // ch
    stripe = ((n + ns - 1) // ns + 7) // 8 * 8
    n_pad = stripe * ns
    mesh = plsc.VectorSubcoreMesh(core_axis_name="c", subcore_axis_name="s")

    @functools.partial(
        pl.kernel,
        mesh=mesh,
        out_type=jax.ShapeDtypeStruct((nc * n_pad, h), jnp.float32),
        scratch_types=[
            pltpu.VMEM((ch,), jnp.int32),
            pltpu.VMEM((ch,), jnp.int32),
            pltpu.VMEM((ch, h), jnp.float32),
            pltpu.VMEM((ch, h), jnp.float32),
            pltpu.SemaphoreType.DMA,
            pltpu.VMEM_SHARED((n_pad, h), jnp.float32),
        ],
    )
    def gc(x_hbm, ef_hbm, esrc_hbm, etgt_hbm, out_hbm,
           esrc_v, etgt_v, rows_v, ef_v, sem, agg_sh):
        c = lax.axis_index("c")
        s = lax.axis_index("s")
        w = s * nc + c

        def zbody(i, carry):
            for q in range(h // 16):
                rows_v[i, pl.ds(q * 16, 16)] = jnp.zeros((16,), jnp.float32)
            return carry

        lax.fori_loop(0, ch, zbody, 0)
        off = 0
        rem = stripe
        while rem > 0:
            ln = min(ch, rem)
            pltpu.sync_copy(rows_v.at[pl.ds(0, ln)],
                            agg_sh.at[pl.ds(s * stripe + off, ln)])
            off += ln
            rem -= ln
        plsc.subcore_barrier()

        nj = (total_chunks - w + nw - 1) // nw

        def body(j, carry):
            base = (w + j * nw) * ch
            pltpu.sync_copy(esrc_hbm.at[pl.ds(base, ch)], esrc_v)
            pltpu.sync_copy(etgt_hbm.at[pl.ds(base, ch)], etgt_v)
            pltpu.async_copy(x_hbm.at[esrc_v], rows_v, sem).wait()
            pltpu.sync_copy(ef_hbm.at[pl.ds(base, ch)], ef_v)

            def mbody(i, mc):
                for q in range(h // 16):
                    sl = pl.ds(q * 16, 16)
                    rows_v[i, sl] = rows_v[i, sl] * ef_v[i, sl]
                return mc

            lax.fori_loop(0, ch, mbody, 0)
            pltpu.sync_copy(rows_v, agg_sh.at[etgt_v], add=True)
            return carry

        lax.fori_loop(0, nj, body, 0)
        plsc.subcore_barrier()

        pltpu.sync_copy(agg_sh.at[pl.ds(s * stripe, stripe)],
                        out_hbm.at[pl.ds(c * n_pad + s * stripe, stripe)])

    return gc, n_pad


def _x0_body(nf_ref, w_ref, b_ref, o_ref):
    o_ref[...] = (jnp.dot(nf_ref[...], w_ref[...],
                          preferred_element_type=jnp.float32) + b_ref[...])


def _ef_body(ed_ref, w_ref, b_ref, o_ref):
    y = jnp.dot(ed_ref[...], w_ref[...],
                preferred_element_type=jnp.float32) + b_ref[...]
    o_ref[...] = jnp.maximum(y, 0.0)


def _upd_body(a_ref, w_ref, b_ref, o_ref, *, n, n_pad, relu):
    ssum = a_ref[:n, :] + a_ref[n_pad:n_pad + n, :]
    y = jnp.dot(ssum, w_ref[...], preferred_element_type=jnp.float32) + b_ref[...]
    o_ref[...] = jnp.maximum(y, 0.0) if relu else y


def _s2s_body(x_ref, bt_ref, wiq_ref, wir_ref, wh_ref, bg_ref, wo_ref, bo_ref,
              o_ref, *, n, h, bsz, steps):
    x = x_ref[...]
    iot = lax.broadcasted_iota(jnp.int32, (n, bsz), 1)
    mf = (bt_ref[...] == iot).astype(jnp.float32)
    wiq = wiq_ref[...]
    wir = wir_ref[...]
    whh = wh_ref[...]
    bg = bg_ref[...]

    z = jnp.zeros((bsz, h), jnp.float32)
    hst, cst, qq, rr = z, z, z, z
    for _ in range(steps):
        gates = (jnp.dot(qq, wiq, preferred_element_type=jnp.float32)
                 + jnp.dot(rr, wir, preferred_element_type=jnp.float32)
                 + jnp.dot(hst, whh, preferred_element_type=jnp.float32) + bg)
        ig = gates[:, :h]
        fg = gates[:, h:2 * h]
        gg = gates[:, 2 * h:3 * h]
        og = gates[:, 3 * h:]
        cst = jax.nn.sigmoid(fg) * cst + jax.nn.sigmoid(ig) * jnp.tanh(gg)
        hst = jax.nn.sigmoid(og) * jnp.tanh(cst)
        hi = lax.Precision.HIGHEST
        qb = jnp.dot(mf, hst, preferred_element_type=jnp.float32, precision=hi)
        ev = jnp.sum(x * qb, axis=1, keepdims=True)
        em = jnp.max(jnp.where(mf > 0.0, ev, -jnp.inf), axis=0, keepdims=True)
        em = jnp.where(jnp.isfinite(em), em, 0.0)
        eb = lax.dot_general(mf, em, (((1,), (1,)), ((), ())),
                             preferred_element_type=jnp.float32, precision=hi)
        av = jnp.exp(ev - eb)
        dn = lax.dot_general(av, mf, (((0,), (0,)), ((), ())),
                             preferred_element_type=jnp.float32, precision=hi)
        db = lax.dot_general(mf, dn, (((1,), (1,)), ((), ())),
                             preferred_element_type=jnp.float32, precision=hi)
        av = av / jnp.maximum(db, 1e-16)
        rr = lax.dot_general(mf * av, x, (((0,), (0,)), ((), ())),
                             preferred_element_type=jnp.float32, precision=hi)
        qq = hst
    o_ref[...] = (jnp.dot(qq, wo_ref[...], preferred_element_type=jnp.float32)
                  + bo_ref[...])


def kernel(node_features, edge_features, Esrc, Etgt, batch,
           W_in, b_in, W_ee, b_ee, Wg0, bg0, Wg1, bg1, Wg2, bg2,
           Wi, Wh, bi, bh, W_out, b_out):
    n, df = node_features.shape
    e, de = edge_features.shape
    h = W_in.shape[1]
    bsz = _NUM_GRAPHS

    x = pl.pallas_call(
        _x0_body,
        out_shape=jax.ShapeDtypeStruct((n, h), jnp.float32),
    )(node_features, W_in, b_in.reshape(1, h))

    eb = 8000
    ef = pl.pallas_call(
        _ef_body,
        grid=(e // eb,),
        in_specs=[
            pl.BlockSpec((eb, de), lambda i: (i, 0)),
            pl.BlockSpec((de, h), lambda i: (0, 0)),
            pl.BlockSpec((1, h), lambda i: (0, 0)),
        ],
        out_specs=pl.BlockSpec((eb, h), lambda i: (i, 0)),
        out_shape=jax.ShapeDtypeStruct((e, h), jnp.float32),
    )(edge_features, W_ee, b_ee.reshape(1, h))

    gc, n_pad = _make_gc_sc(n, e, h)
    layers = ((Wg0, bg0, True), (Wg1, bg1, True), (Wg2, bg2, False))
    for wg, bgl, relu in layers:
        agg2 = gc(x, ef, Esrc, Etgt)
        x = pl.pallas_call(
            functools.partial(_upd_body, n=n, n_pad=n_pad, relu=relu),
            out_shape=jax.ShapeDtypeStruct((n, h), jnp.float32),
        )(agg2, wg, bgl.reshape(1, h))

    out = pl.pallas_call(
        functools.partial(_s2s_body, n=n, h=h, bsz=bsz, steps=_STEPS),
        out_shape=jax.ShapeDtypeStruct((bsz, 1), jnp.float32),
    )(x, batch.reshape(n, 1), Wi[:h], Wi[h:], Wh,
      (bi + bh).reshape(1, 4 * h), W_out, b_out.reshape(1, 1))
    return out
